# Initial kernel scaffold; baseline (speedup 1.0000x reference)
import jax
import jax.numpy as jnp
from jax.experimental import pallas as pl


def kernel(x, edge_index, W1, b1, W2, b2, W3, b3):
    raise NotImplementedError("write your pallas kernel here")


# speedup vs baseline: 21.5978x; 21.5978x over previous
import functools

import jax
import jax.numpy as jnp
from jax import lax
from jax.experimental import pallas as pl
from jax.experimental.pallas import tpu as pltpu
from jax.experimental.pallas import tpu_sc as plsc

N = 10000
E = 320000
NC = 2
NS = 16
NW = NC * NS
C = 80
K = E // (NW * C)
NPAD = 10240
RPT = NPAD // NS
ZR = 128
BM = 1000


def _mesh():
    return plsc.VectorSubcoreMesh(core_axis_name="c", subcore_axis_name="s")


@functools.partial(
    pl.kernel,
    out_type=jax.ShapeDtypeStruct((NC, 1, NPAD), jnp.float32),
    mesh=_mesh(),
    scratch_types=[pltpu.VMEM_SHARED((NPAD,), jnp.float32)],
    compiler_params=pltpu.CompilerParams(use_tc_tiling_on_sc=False),
)
def _deg_pass(dstr_hbm, out_hbm, acc):
    c = lax.axis_index("c")
    s = lax.axis_index("s")
    w = s * NC + c

    def body(idx_d, ones, zbuf):
        one = jnp.ones((16,), jnp.float32)
        zero = jnp.zeros((16,), jnp.float32)
        for i in range(C // 16):
            ones[pl.ds(i * 16, 16)] = one

        @pl.loop(0, RPT // 16)
        def _(i):
            zbuf[pl.ds(i * 16, 16)] = zero

        pltpu.sync_copy(zbuf, acc.at[pl.ds(s * RPT, RPT)])
        pltpu.sync_copy(dstr_hbm.at[w], idx_d)
        plsc.subcore_barrier()

        @pl.loop(0, K)
        def _(j):
            pltpu.sync_copy(ones, acc.at[idx_d.at[j]], add=True)

        plsc.subcore_barrier()
        pltpu.sync_copy(acc.at[pl.ds(s * RPT, RPT)],
                        out_hbm.at[c, 0, pl.ds(s * RPT, RPT)])

    pl.run_scoped(
        body,
        pltpu.VMEM((K, C), jnp.int32),
        pltpu.VMEM((C,), jnp.float32),
        pltpu.VMEM((RPT,), jnp.float32),
    )


def _make_edge_pass(F):
    @functools.partial(
        pl.kernel,
        out_type=jax.ShapeDtypeStruct((NC, NPAD, F), jnp.float32),
        mesh=_mesh(),
        scratch_types=[pltpu.VMEM_SHARED((NPAD, F), jnp.float32)],
        compiler_params=pltpu.CompilerParams(use_tc_tiling_on_sc=(F == 128)),
    )
    def edge_pass(y_hbm, srcr_hbm, dstr_hbm, out_hbm, acc):
        c = lax.axis_index("c")
        s = lax.axis_index("s")
        w = s * NC + c

        def body(idx_s, idx_d, rows, sem):
            zero = jnp.zeros((16,), jnp.float32)

            @pl.loop(0, C)
            def _(i):
                for f in range(F // 16):
                    rows[i, pl.ds(f * 16, 16)] = zero

            for m in range(RPT // C):
                pltpu.sync_copy(rows, acc.at[pl.ds(s * RPT + m * C, C)])
            pltpu.sync_copy(srcr_hbm.at[w], idx_s)
            pltpu.sync_copy(dstr_hbm.at[w], idx_d)
            plsc.subcore_barrier()

            @pl.loop(0, K)
            def _(j):
                pltpu.async_copy(y_hbm.at[idx_s.at[j]], rows, sem).wait()
                pltpu.sync_copy(rows, acc.at[idx_d.at[j]], add=True)

            plsc.subcore_barrier()
            pltpu.sync_copy(acc.at[pl.ds(s * RPT, RPT)],
                            out_hbm.at[c, pl.ds(s * RPT, RPT)])

        pl.run_scoped(
            body,
            pltpu.VMEM((K, C), jnp.int32),
            pltpu.VMEM((K, C), jnp.int32),
            pltpu.VMEM((C, F), jnp.float32),
            pltpu.SemaphoreType.DMA,
        )

    return edge_pass


_edge_pass_128 = _make_edge_pass(128)
_edge_pass_64 = _make_edge_pass(64)


@functools.partial(
    pl.kernel,
    out_type=jax.ShapeDtypeStruct((NC, 1, NPAD), jnp.float32),
    mesh=_mesh(),
    scratch_types=[pltpu.VMEM_SHARED((NPAD,), jnp.float32)],
    compiler_params=pltpu.CompilerParams(use_tc_tiling_on_sc=False),
)
def _edge_pass_1(y_hbm, srcr_hbm, dstr_hbm, out_hbm, acc):
    c = lax.axis_index("c")
    s = lax.axis_index("s")
    w = s * NC + c

    def body(idx_s, idx_d, vals, zbuf, sem):
        zero = jnp.zeros((16,), jnp.float32)

        @pl.loop(0, RPT // 16)
        def _(i):
            zbuf[pl.ds(i * 16, 16)] = zero

        pltpu.sync_copy(zbuf, acc.at[pl.ds(s * RPT, RPT)])
        pltpu.sync_copy(srcr_hbm.at[w], idx_s)
        pltpu.sync_copy(dstr_hbm.at[w], idx_d)
        plsc.subcore_barrier()

        @pl.loop(0, K)
        def _(j):
            pltpu.async_copy(y_hbm.at[idx_s.at[j]], vals, sem).wait()
            pltpu.sync_copy(vals, acc.at[idx_d.at[j]], add=True)

        plsc.subcore_barrier()
        pltpu.sync_copy(acc.at[pl.ds(s * RPT, RPT)],
                        out_hbm.at[c, 0, pl.ds(s * RPT, RPT)])

    pl.run_scoped(
        body,
        pltpu.VMEM((K, C), jnp.int32),
        pltpu.VMEM((K, C), jnp.int32),
        pltpu.VMEM((C,), jnp.float32),
        pltpu.VMEM((RPT,), jnp.float32),
        pltpu.SemaphoreType.DMA,
    )


def _mm1_body(x_ref, w_ref, o_ref):
    o_ref[...] = jnp.dot(x_ref[...], w_ref[...],
                         preferred_element_type=jnp.float32)


def _mm1(x, W1):
    return pl.pallas_call(
        _mm1_body,
        grid=(N // BM,),
        in_specs=[pl.BlockSpec((BM, 128), lambda i: (i, 0)),
                  pl.BlockSpec((128, 128), lambda i: (0, 0))],
        out_specs=pl.BlockSpec((BM, 128), lambda i: (i, 0)),
        out_shape=jax.ShapeDtypeStruct((N, 128), jnp.float32),
    )(x, W1)


def _scale1_body(xw_ref, d0_ref, d1_ref, y_ref, dis_ref):
    dis = lax.rsqrt(d0_ref[...] + d1_ref[...] + 1.0)
    y_ref[...] = xw_ref[...] * dis
    dis_ref[...] = dis


def _scale1(xw, d0, d1):
    return pl.pallas_call(
        _scale1_body,
        grid=(N // BM,),
        in_specs=[pl.BlockSpec((BM, 128), lambda i: (i, 0)),
                  pl.BlockSpec((BM, 1), lambda i: (i, 0)),
                  pl.BlockSpec((BM, 1), lambda i: (i, 0))],
        out_specs=[pl.BlockSpec((BM, 128), lambda i: (i, 0)),
                   pl.BlockSpec((BM, 1), lambda i: (i, 0))],
        out_shape=[jax.ShapeDtypeStruct((N, 128), jnp.float32),
                   jax.ShapeDtypeStruct((N, 1), jnp.float32)],
    )(xw, d0, d1)


def _make_comb(Fin, Fout):
    def body(p_ref, y_ref, dis_ref, b_ref, w_ref, o_ref):
        h = jnp.tanh((p_ref[0] + p_ref[1] + y_ref[...]) * dis_ref[...]
                     + b_ref[...])
        o_ref[...] = jnp.dot(h, w_ref[...],
                             preferred_element_type=jnp.float32) * dis_ref[...]

    def comb(p, y, dis, b, w):
        return pl.pallas_call(
            body,
            grid=(N // BM,),
            in_specs=[pl.BlockSpec((NC, BM, Fin), lambda i: (0, i, 0)),
                      pl.BlockSpec((BM, Fin), lambda i: (i, 0)),
                      pl.BlockSpec((BM, 1), lambda i: (i, 0)),
                      pl.BlockSpec((1, Fin), lambda i: (0, 0)),
                      pl.BlockSpec((Fin, Fout), lambda i: (0, 0))],
            out_specs=pl.BlockSpec((BM, Fout), lambda i: (i, 0)),
            out_shape=jax.ShapeDtypeStruct((N, Fout), jnp.float32),
        )(p, y, dis, b, w)

    return comb


_comb1 = _make_comb(128, 64)
_comb2 = _make_comb(64, 1)


def _comb3_body(pa_ref, pb_ref, y_ref, dis_ref, b_ref, o_ref):
    o_ref[...] = ((pa_ref[...] + pb_ref[...] + y_ref[...]) * dis_ref[...]
                  + b_ref[...])


def _comb3(pa, pb, y, dis, b):
    return pl.pallas_call(
        _comb3_body,
        grid=(N // BM,),
        in_specs=[pl.BlockSpec((BM, 1), lambda i: (i, 0)),
                  pl.BlockSpec((BM, 1), lambda i: (i, 0)),
                  pl.BlockSpec((BM, 1), lambda i: (i, 0)),
                  pl.BlockSpec((BM, 1), lambda i: (i, 0)),
                  pl.BlockSpec((1, 1), lambda i: (0, 0))],
        out_specs=pl.BlockSpec((BM, 1), lambda i: (i, 0)),
        out_shape=jax.ShapeDtypeStruct((N, 1), jnp.float32),
    )(pa, pb, y, dis, b)


def kernel(x, edge_index, W1, b1, W2, b2, W3, b3):
    src = edge_index[0].reshape(NW, K, C)
    dst = edge_index[1].reshape(NW, K, C)

    degp = _deg_pass(dst)
    xw1 = _mm1(x, W1)
    d0 = degp[0, 0, :N].reshape(N, 1)
    d1 = degp[1, 0, :N].reshape(N, 1)
    y1, dis = _scale1(xw1, d0, d1)

    p1 = _edge_pass_128(y1, src, dst)
    y2 = _comb1(p1, y1, dis, b1.reshape(1, 128), W2)

    p2 = _edge_pass_64(y2, src, dst)
    y3 = _comb2(p2, y2, dis, b2.reshape(1, 64), W3)

    p3 = _edge_pass_1(y3.reshape(N), src, dst)
    q0 = p3[0, 0, :N].reshape(N, 1)
    q1 = p3[1, 0, :N].reshape(N, 1)
    return _comb3(q0, q1, y3, dis, b3.reshape(1, 1))
